# Initial kernel scaffold; baseline (speedup 1.0000x reference)
import jax
import jax.numpy as jnp
from jax.experimental import pallas as pl


def kernel(x, edge_index, W, b):
    raise NotImplementedError("write your pallas kernel here")


# speedup vs baseline: 33.4849x; 33.4849x over previous
import functools

import jax
import jax.numpy as jnp
from jax import lax
from jax.experimental import pallas as pl
from jax.experimental.pallas import tpu as pltpu
from jax.experimental.pallas import tpu_sc as plsc

N_NODES = 10000
N_EDGES = 160000
D = 256

NW = 32
E_PER_W = 5008
E_PAD = NW * E_PER_W
N_PAD = 10016


def _matvec_tc(x, w):
    def body(x_ref, w_ref, o_ref):
        o_ref[...] = lax.dot_general(
            x_ref[...], w_ref[...], (((1,), (0,)), ((), ())),
            preferred_element_type=jnp.float32)

    return pl.pallas_call(
        body,
        grid=(5,),
        in_specs=[pl.BlockSpec((2000, D), lambda i: (i, 0)),
                  pl.BlockSpec((D, 1), lambda i: (0, 0))],
        out_specs=pl.BlockSpec((2000, 1), lambda i: (i, 0)),
        out_shape=jax.ShapeDtypeStruct((N_NODES, 1), jnp.float32),
    )(x, w)


def _combine_tc(parts, bias):
    def body(p_ref, b_ref, o_ref):
        o_ref[...] = jnp.sum(p_ref[...], axis=0, keepdims=True) + b_ref[0]

    return pl.pallas_call(
        body,
        out_shape=jax.ShapeDtypeStruct((1, N_PAD), jnp.float32),
    )(parts, bias)


@functools.cache
def _make_segsum_sc():
    mesh = plsc.VectorSubcoreMesh(core_axis_name="c", subcore_axis_name="s",
                                  num_cores=2, num_subcores=16)

    @functools.partial(
        pl.kernel,
        mesh=mesh,
        compiler_params=pltpu.CompilerParams(needs_layout_passes=False),
        out_type=jax.ShapeDtypeStruct((NW, N_PAD), jnp.float32),
        scratch_types=[
            pltpu.VMEM((E_PER_W,), jnp.int32),
            pltpu.VMEM((E_PER_W,), jnp.int32),
            pltpu.VMEM((N_PAD,), jnp.float32),
            pltpu.VMEM((N_PAD,), jnp.float32),
        ],
    )
    def segsum(y_hbm, src_hbm, dst_hbm, out_hbm, src_v, dst_v, y_v, z_v):
        wid = lax.axis_index("s") * 2 + lax.axis_index("c")
        base = wid * E_PER_W
        pltpu.sync_copy(src_hbm.at[pl.ds(base, E_PER_W)], src_v)
        pltpu.sync_copy(dst_hbm.at[pl.ds(base, E_PER_W)], dst_v)
        pltpu.sync_copy(y_hbm, y_v)

        def zero_body(i, carry):
            z_v[pl.ds(i * 16, 16)] = jnp.zeros((16,), jnp.float32)
            return carry
        lax.fori_loop(0, N_PAD // 16, zero_body, 0)

        def edge_body(i, carry):
            s16 = src_v[pl.ds(i * 16, 16)]
            d16 = dst_v[pl.ds(i * 16, 16)]
            vals = plsc.load_gather(y_v, [s16])
            plsc.addupdate_scatter(z_v, [d16], vals)
            return carry
        lax.fori_loop(0, E_PER_W // 16, edge_body, 0)

        pltpu.sync_copy(z_v, out_hbm.at[wid])

    return segsum


def kernel(x, edge_index, W, b):
    _segsum_sc = _make_segsum_sc()
    src = edge_index[0].astype(jnp.int32)
    dst = edge_index[1].astype(jnp.int32)
    pad = jnp.full((E_PAD - N_EDGES,), N_NODES, jnp.int32)
    src = jnp.concatenate([src, pad])
    dst = jnp.concatenate([dst, pad])

    y = _matvec_tc(x, W)
    y_pad = jnp.pad(y[:, 0], (0, N_PAD - N_NODES))

    parts1 = _segsum_sc(y_pad, src, dst)
    z1 = _combine_tc(parts1, jnp.zeros((1,), jnp.float32))[0]

    parts2 = _segsum_sc(z1, src, dst)
    z2 = _combine_tc(parts2, b)

    return z2[0, :N_NODES, None]


# speedup vs baseline: 37.7822x; 1.1283x over previous
import functools

import jax
import jax.numpy as jnp
from jax import lax
from jax.experimental import pallas as pl
from jax.experimental.pallas import tpu as pltpu
from jax.experimental.pallas import tpu_sc as plsc

N_NODES = 10000
N_EDGES = 160000
D = 256

NW = 32
E_PER_W = 5008
E_PAD = NW * E_PER_W
N_PAD = 10016


def _matvec_tc(x, w):
    def body(x_ref, w_ref, o_ref):
        o_ref[...] = lax.dot_general(
            x_ref[...], w_ref[...], (((1,), (0,)), ((), ())),
            preferred_element_type=jnp.float32)

    return pl.pallas_call(
        body,
        grid=(5,),
        in_specs=[pl.BlockSpec((2000, D), lambda i: (i, 0)),
                  pl.BlockSpec((D, 1), lambda i: (0, 0))],
        out_specs=pl.BlockSpec((2000, 1), lambda i: (i, 0)),
        out_shape=jax.ShapeDtypeStruct((N_NODES, 1), jnp.float32),
    )(x, w)


def _combine_tc(parts, bias):
    def body(p_ref, b_ref, o_ref):
        o_ref[...] = jnp.sum(p_ref[...], axis=0, keepdims=True) + b_ref[0]

    return pl.pallas_call(
        body,
        out_shape=jax.ShapeDtypeStruct((1, N_PAD), jnp.float32),
    )(parts, bias)


@functools.cache
def _make_segsum_sc():
    mesh = plsc.VectorSubcoreMesh(core_axis_name="c", subcore_axis_name="s",
                                  num_cores=2, num_subcores=16)

    @functools.partial(
        pl.kernel,
        mesh=mesh,
        compiler_params=pltpu.CompilerParams(needs_layout_passes=False),
        out_type=jax.ShapeDtypeStruct((NW, N_PAD), jnp.float32),
        scratch_types=[
            pltpu.VMEM((E_PER_W,), jnp.int32),
            pltpu.VMEM((E_PER_W,), jnp.int32),
            pltpu.VMEM((N_PAD,), jnp.float32),
            pltpu.VMEM((N_PAD,), jnp.float32),
        ],
    )
    def segsum(y_hbm, src_hbm, dst_hbm, out_hbm, src_v, dst_v, y_v, z_v):
        wid = lax.axis_index("s") * 2 + lax.axis_index("c")
        base = wid * E_PER_W
        pltpu.sync_copy(src_hbm.at[pl.ds(base, E_PER_W)], src_v)
        pltpu.sync_copy(dst_hbm.at[pl.ds(base, E_PER_W)], dst_v)
        pltpu.sync_copy(y_hbm, y_v)

        @plsc.parallel_loop(0, N_PAD, 16, unroll=8)
        def _zero(i):
            z_v[pl.ds(i, 16)] = jnp.zeros((16,), jnp.float32)

        @plsc.parallel_loop(0, E_PER_W, 16, unroll=8)
        def _edges(i):
            s16 = src_v[pl.ds(i, 16)]
            d16 = dst_v[pl.ds(i, 16)]
            vals = plsc.load_gather(y_v, [s16])
            plsc.addupdate_scatter(z_v, [d16], vals)

        pltpu.sync_copy(z_v, out_hbm.at[wid])

    return segsum


def kernel(x, edge_index, W, b):
    _segsum_sc = _make_segsum_sc()
    src = edge_index[0].astype(jnp.int32)
    dst = edge_index[1].astype(jnp.int32)
    pad = jnp.full((E_PAD - N_EDGES,), N_NODES, jnp.int32)
    src = jnp.concatenate([src, pad])
    dst = jnp.concatenate([dst, pad])

    y = _matvec_tc(x, W)
    y_pad = jnp.pad(y[:, 0], (0, N_PAD - N_NODES))

    parts1 = _segsum_sc(y_pad, src, dst)
    z1 = _combine_tc(parts1, jnp.zeros((1,), jnp.float32))[0]

    parts2 = _segsum_sc(z1, src, dst)
    z2 = _combine_tc(parts2, b)

    return z2[0, :N_NODES, None]
